# baseline (speedup 1.0000x reference)
import functools

import jax
import jax.numpy as jnp
from jax import lax
from jax.experimental import pallas as pl
from jax.experimental.pallas import tpu as pltpu
from jax.experimental.pallas import tpu_sc as plsc

LAMBDA = 0.0001
B = 16384
D = 32
NC, NS, L = 2, 16, 16
NW = NC * NS
BPW = B // NW
CHUNK = 128
NCHUNK = BPW // CHUNK
GROUPS = BPW // L


def _sc_body(user_hbm, itemi_hbm, itemj_hbm, eu_hbm, ei_hbm,
             x_hbm, sums_hbm,
             uidx, iidx, jidx, ubuf, ibuf, jbuf, xbuf, sbuf, sem):
    wid = lax.axis_index("s") * NC + lax.axis_index("c")
    base = wid * BPW

    pltpu.sync_copy(user_hbm.at[pl.ds(base, BPW)], uidx)
    pltpu.sync_copy(itemi_hbm.at[pl.ds(base, BPW)], iidx)
    pltpu.sync_copy(itemj_hbm.at[pl.ds(base, BPW)], jidx)

    copies = []
    for c in range(NCHUNK):
        sl = pl.ds(c * CHUNK, CHUNK)
        copies.append(pltpu.async_copy(eu_hbm.at[uidx.at[sl]], ubuf.at[sl], sem))
        copies.append(pltpu.async_copy(ei_hbm.at[iidx.at[sl]], ibuf.at[sl], sem))
        copies.append(pltpu.async_copy(ei_hbm.at[jidx.at[sl]], jbuf.at[sl], sem))
    for cp in copies:
        cp.wait()

    lanes = lax.iota(jnp.int32, L)
    zeros = jnp.zeros((L,), jnp.float32)

    def group(g, carry):
        su, si, sj = carry
        rvec = g * L + lanes
        acc_i = zeros
        acc_j = zeros
        for d in range(D):
            dvec = jnp.full((L,), d, jnp.int32)
            uu = plsc.load_gather(ubuf, [rvec, dvec])
            vi = plsc.load_gather(ibuf, [rvec, dvec])
            vj = plsc.load_gather(jbuf, [rvec, dvec])
            acc_i = acc_i + uu * vi
            acc_j = acc_j + uu * vj
            su = su + uu * uu
            si = si + vi * vi
            sj = sj + vj * vj
        xbuf[pl.ds(g * L, L)] = acc_i - acc_j
        return su, si, sj

    su, si, sj = lax.fori_loop(0, GROUPS, group, (zeros, zeros, zeros))
    sbuf[pl.ds(0, L)] = su
    sbuf[pl.ds(L, L)] = si
    sbuf[pl.ds(2 * L, L)] = sj
    pltpu.sync_copy(xbuf, x_hbm.at[pl.ds(base, BPW)])
    pltpu.sync_copy(sbuf, sums_hbm.at[pl.ds(wid * 3 * L, 3 * L)])


_sc_gather_dots = functools.partial(
    pl.kernel,
    out_type=[jax.ShapeDtypeStruct((B,), jnp.float32),
              jax.ShapeDtypeStruct((NW * 3 * L,), jnp.float32)],
    mesh=plsc.VectorSubcoreMesh(core_axis_name="c", subcore_axis_name="s"),
    compiler_params=pltpu.CompilerParams(
        needs_layout_passes=False, use_tc_tiling_on_sc=False),
    scratch_types=[
        pltpu.VMEM((BPW,), jnp.int32),
        pltpu.VMEM((BPW,), jnp.int32),
        pltpu.VMEM((BPW,), jnp.int32),
        pltpu.VMEM((BPW, D), jnp.float32),
        pltpu.VMEM((BPW, D), jnp.float32),
        pltpu.VMEM((BPW, D), jnp.float32),
        pltpu.VMEM((BPW,), jnp.float32),
        pltpu.VMEM((3 * L,), jnp.float32),
        pltpu.SemaphoreType.DMA,
    ],
)(_sc_body)


def _tc_body(x_ref, s_ref, o_ref):
    x = x_ref[...]
    sp = jnp.maximum(-x, 0.0) + jnp.log1p(jnp.exp(-jnp.abs(x)))
    l2 = LAMBDA * jnp.sum(s_ref[...]) / (B * D)
    o_ref[0, 0] = jnp.sum(sp) / B + l2


_tc_loss = pl.pallas_call(
    _tc_body,
    out_shape=jax.ShapeDtypeStruct((1, 1), jnp.float32),
    in_specs=[pl.BlockSpec((128, 128), lambda: (0, 0)),
              pl.BlockSpec((16, 96), lambda: (0, 0))],
    out_specs=pl.BlockSpec(memory_space=pltpu.SMEM),
)


def kernel(user, item_i, item_j, embed_user, embed_item):
    x, sums = _sc_gather_dots(user, item_i, item_j, embed_user, embed_item)
    out = _tc_loss(x.reshape(128, 128), sums.reshape(16, 96))
    return out[0, 0]
